# baseline (speedup 1.0000x reference)
import jax
import jax.numpy as jnp
from jax import lax
from jax.experimental import pallas as pl
from jax.experimental.pallas import tpu as pltpu
from jax.experimental.pallas import tpu_sc as plsc

N = 100000
E = 3200000
D_IN = 5
H = 32
HH = 16

ROWW = 100
NROWS = E // ROWW
NS = 16
CHUNK = 8
DPT = NROWS // 32
DCH = DPT // CHUNK
MICRO = 200
SROWS = E // MICRO
SPT = SROWS // NS
IDXB = 40
NOUT = SPT // IDXB
NCHI = IDXB
ACC_SLICE = 6256
ACC_LAST = N - 15 * ACC_SLICE
WOUT = 112


def _acc_chunks(total):
    offs = []
    o = 0
    while o < total:
        sz = min(WOUT, total - o)
        offs.append((o, sz))
        o += sz
    return offs
DEG_SLICE = 6240
DEG_LAST = N - 15 * DEG_SLICE

B = 2000
NB = N // B


def _sc_deg_body(col2, deg0, deg1, colv, onesv, dbuf, dacc):
    cid = lax.axis_index("c")
    sid = lax.axis_index("s")

    def fill_ones(i, c):
        onesv[pl.ds(i * 16, 16)] = jnp.full((16,), 1.0, jnp.float32)
        return c

    lax.fori_loop(0, 7, fill_ones, 0)

    def fill_zero(i, c):
        dbuf[pl.ds(i * 16, 16)] = jnp.zeros((16,), jnp.float32)
        return c

    lax.fori_loop(0, DEG_LAST // 16, fill_zero, 0)

    @pl.when(sid < 15)
    def _():
        pltpu.sync_copy(dbuf.at[pl.ds(0, DEG_SLICE)],
                        dacc.at[pl.ds(sid * DEG_SLICE, DEG_SLICE)])

    @pl.when(sid == 15)
    def _():
        pltpu.sync_copy(dbuf, dacc.at[pl.ds(15 * DEG_SLICE, DEG_LAST)])

    plsc.subcore_barrier()

    w = cid * NS + sid

    def chunk_body(ch, c):
        base = w * DPT + ch * CHUNK
        pltpu.sync_copy(col2.at[pl.ds(base, CHUNK)], colv)
        for j in range(CHUNK):
            pltpu.sync_copy(onesv.at[pl.ds(0, ROWW)],
                            dacc.at[colv.at[j]], add=True)
        return c

    lax.fori_loop(0, DCH, chunk_body, 0)
    plsc.subcore_barrier()

    def writeout(dst):
        @pl.when(sid < 15)
        def _():
            pltpu.sync_copy(dacc.at[pl.ds(sid * DEG_SLICE, DEG_SLICE)],
                            dbuf.at[pl.ds(0, DEG_SLICE)])
            pltpu.sync_copy(dbuf.at[pl.ds(0, DEG_SLICE)],
                            dst.at[pl.ds(sid * DEG_SLICE, DEG_SLICE)])

        @pl.when(sid == 15)
        def _():
            pltpu.sync_copy(dacc.at[pl.ds(15 * DEG_SLICE, DEG_LAST)], dbuf)
            pltpu.sync_copy(dbuf, dst.at[pl.ds(15 * DEG_SLICE, DEG_LAST)])

    @pl.when(cid == 0)
    def _():
        writeout(deg0)

    @pl.when(cid == 1)
    def _():
        writeout(deg1)


def _sc_spmm_body(row2, col2, g0, g1, s0, s1, rowv, colv, rows2, wbuf, acc,
                  sg0, sg1, ss0, ss1):
    cid = lax.axis_index("c")
    sid = lax.axis_index("s")

    def fill_zero(i, c):
        wbuf[i, :] = jnp.zeros((16,), jnp.float32)
        return c

    lax.fori_loop(0, WOUT, fill_zero, 0)

    base = sid * ACC_SLICE

    @pl.when(sid < 15)
    def _():
        for o, sz in _acc_chunks(ACC_SLICE):
            pltpu.sync_copy(wbuf.at[pl.ds(0, sz)],
                            acc.at[pl.ds(base + o, sz)])

    @pl.when(sid == 15)
    def _():
        for o, sz in _acc_chunks(ACC_LAST):
            pltpu.sync_copy(wbuf.at[pl.ds(0, sz)],
                            acc.at[pl.ds(base + o, sz)])

    plsc.subcore_barrier()

    sg = (sg0, sg1)
    ss = (ss0, ss1)

    def fire_g(gref, c, b):
        pltpu.async_copy(gref.at[rowv.at[c]], rows2.at[b], sg[b])

    def drain_g(gref, b):
        pltpu.make_async_copy(gref.at[pl.ds(0, MICRO)],
                              rows2.at[b], sg[b]).wait()

    def fire_s(c, b):
        pltpu.async_copy(rows2.at[b], acc.at[colv.at[c]], ss[b], add=True)

    def drain_s(b):
        pltpu.make_async_copy(rows2.at[b],
                              acc.at[pl.ds(0, MICRO)], ss[b]).wait()

    def outer(t, c_):
        obase = sid * SPT + t * IDXB
        pltpu.sync_copy(row2.at[pl.ds(obase, IDXB)], rowv)
        pltpu.sync_copy(col2.at[pl.ds(obase, IDXB)], colv)

        def run(gref):
            fire_g(gref, 0, 0)
            for c in range(NCHI):
                b = c % 2
                if c >= 1:
                    drain_s(1 - b)
                if c + 1 < NCHI:
                    fire_g(gref, c + 1, 1 - b)
                drain_g(gref, b)
                fire_s(c, b)
            drain_s((NCHI - 1) % 2)

        @pl.when(cid == 0)
        def _():
            run(g0)

        @pl.when(cid == 1)
        def _():
            run(g1)
        return c_

    lax.fori_loop(0, NOUT, outer, 0)
    plsc.subcore_barrier()

    def writeout(dst):
        @pl.when(sid < 15)
        def _():
            for o, sz in _acc_chunks(ACC_SLICE):
                pltpu.sync_copy(acc.at[pl.ds(base + o, sz)],
                                wbuf.at[pl.ds(0, sz)])
                pltpu.sync_copy(wbuf.at[pl.ds(0, sz)],
                                dst.at[pl.ds(base + o, sz)])

        @pl.when(sid == 15)
        def _():
            for o, sz in _acc_chunks(ACC_LAST):
                pltpu.sync_copy(acc.at[pl.ds(base + o, sz)],
                                wbuf.at[pl.ds(0, sz)])
                pltpu.sync_copy(wbuf.at[pl.ds(0, sz)],
                                dst.at[pl.ds(base + o, sz)])

    @pl.when(cid == 0)
    def _():
        writeout(s0)

    @pl.when(cid == 1)
    def _():
        writeout(s1)


def _sc_mesh():
    return plsc.VectorSubcoreMesh(core_axis_name="c", subcore_axis_name="s")


_SC_PARAMS = pltpu.CompilerParams(use_tc_tiling_on_sc=False)


def _deg_call(col2):
    f = pl.kernel(
        _sc_deg_body,
        out_type=(jax.ShapeDtypeStruct((N,), jnp.float32),
                  jax.ShapeDtypeStruct((N,), jnp.float32)),
        mesh=_sc_mesh(),
        compiler_params=_SC_PARAMS,
        scratch_types=[
            pltpu.VMEM((CHUNK, ROWW), jnp.int32),
            pltpu.VMEM((112,), jnp.float32),
            pltpu.VMEM((DEG_LAST,), jnp.float32),
            pltpu.VMEM_SHARED((N,), jnp.float32),
        ],
    )
    return f(col2)


def _spmm_call(row2, col2, g0, g1):
    f = pl.kernel(
        _sc_spmm_body,
        out_type=(jax.ShapeDtypeStruct((N, HH), jnp.float32),
                  jax.ShapeDtypeStruct((N, HH), jnp.float32)),
        mesh=_sc_mesh(),
        compiler_params=_SC_PARAMS,
        scratch_types=[
            pltpu.VMEM((IDXB, MICRO), jnp.int32),
            pltpu.VMEM((IDXB, MICRO), jnp.int32),
            pltpu.VMEM((2, MICRO, HH), jnp.float32),
            pltpu.VMEM((WOUT, HH), jnp.float32),
            pltpu.VMEM_SHARED((N, HH), jnp.float32),
            pltpu.SemaphoreType.DMA,
            pltpu.SemaphoreType.DMA,
            pltpu.SemaphoreType.DMA,
            pltpu.SemaphoreType.DMA,
        ],
    )
    return f(row2, col2, g0, g1)


def _tc1_body(x_ref, w1_ref, d0_ref, d1_ref, g0_ref, g1_ref, dinv_ref):
    deg = d0_ref[...] + d1_ref[...]
    dinv = jnp.where(deg > 0.0, lax.rsqrt(jnp.maximum(deg, 1e-12)), 0.0)
    g = jnp.dot(x_ref[...], w1_ref[...],
                preferred_element_type=jnp.float32) * dinv
    g0_ref[...] = g[:, :HH]
    g1_ref[...] = g[:, HH:]
    dinv_ref[...] = dinv


def _tc1(x, W1, d0, d1):
    return pl.pallas_call(
        _tc1_body,
        grid=(NB,),
        in_specs=[
            pl.BlockSpec((B, D_IN), lambda i: (i, 0)),
            pl.BlockSpec((D_IN, H), lambda i: (0, 0)),
            pl.BlockSpec((B, 1), lambda i: (i, 0)),
            pl.BlockSpec((B, 1), lambda i: (i, 0)),
        ],
        out_specs=[
            pl.BlockSpec((B, HH), lambda i: (i, 0)),
            pl.BlockSpec((B, HH), lambda i: (i, 0)),
            pl.BlockSpec((B, 1), lambda i: (i, 0)),
        ],
        out_shape=[
            jax.ShapeDtypeStruct((N, HH), jnp.float32),
            jax.ShapeDtypeStruct((N, HH), jnp.float32),
            jax.ShapeDtypeStruct((N, 1), jnp.float32),
        ],
    )(x, W1, d0, d1)


def _tcmid_body(s0_ref, s1_ref, dinv_ref, b_ref, w_ref, g0_ref, g1_ref):
    dinv = dinv_ref[...]
    h = jnp.concatenate([s0_ref[...], s1_ref[...]], axis=1) * dinv + b_ref[...]
    h = jnp.maximum(h, 0.0)
    g = jnp.dot(h, w_ref[...], preferred_element_type=jnp.float32) * dinv
    g0_ref[...] = g[:, :HH]
    g1_ref[...] = g[:, HH:]


def _tcmid(s0, s1, dinv, b, W):
    return pl.pallas_call(
        _tcmid_body,
        grid=(NB,),
        in_specs=[
            pl.BlockSpec((B, HH), lambda i: (i, 0)),
            pl.BlockSpec((B, HH), lambda i: (i, 0)),
            pl.BlockSpec((B, 1), lambda i: (i, 0)),
            pl.BlockSpec((1, H), lambda i: (0, 0)),
            pl.BlockSpec((H, H), lambda i: (0, 0)),
        ],
        out_specs=[
            pl.BlockSpec((B, HH), lambda i: (i, 0)),
            pl.BlockSpec((B, HH), lambda i: (i, 0)),
        ],
        out_shape=[
            jax.ShapeDtypeStruct((N, HH), jnp.float32),
            jax.ShapeDtypeStruct((N, HH), jnp.float32),
        ],
    )(s0, s1, dinv, b, W)


def _heads_body(s0_ref, s1_ref, dinv_ref, b3_ref, wd_ref, bd_ref, wa_ref,
                ba_ref, wt_ref, bt_ref, wact_ref, bact_ref,
                dist_ref, att_ref, tls_ref, al_ref, best_val, best_row):
    i = pl.program_id(0)

    @pl.when(i == 0)
    def _():
        best_val[0] = -jnp.inf

    dinv = dinv_ref[...]
    h = jnp.concatenate([s0_ref[...], s1_ref[...]], axis=1) * dinv + b3_ref[...]
    h = jnp.maximum(h, 0.0)
    d = jnp.dot(h, wd_ref[...], preferred_element_type=jnp.float32) + bd_ref[...]
    a = jnp.dot(h, wa_ref[...], preferred_element_type=jnp.float32) + ba_ref[...]
    dist_ref[...] = d
    att_ref[...] = a

    loc_max = jnp.max(a)
    ids = lax.broadcasted_iota(jnp.int32, (B, 1), 0)
    loc_arg = jnp.min(jnp.where(a == loc_max, ids, N))

    @pl.when(loc_max > best_val[0])
    def _():
        best_val[0] = loc_max
        best_row[...] = jnp.sum(h * (ids == loc_arg).astype(jnp.float32),
                                axis=0, keepdims=True)

    @pl.when(i == NB - 1)
    def _():
        ht = best_row[...]
        tls_ref[...] = jnp.dot(ht, wt_ref[...],
                               preferred_element_type=jnp.float32) + bt_ref[...]
        al_ref[...] = jnp.dot(ht, wact_ref[...],
                              preferred_element_type=jnp.float32) + bact_ref[...]


def _heads(s0, s1, dinv, b3, Wd, bd, Wa, ba, Wt, bt, Wact, bact):
    return pl.pallas_call(
        _heads_body,
        grid=(NB,),
        in_specs=[
            pl.BlockSpec((B, HH), lambda i: (i, 0)),
            pl.BlockSpec((B, HH), lambda i: (i, 0)),
            pl.BlockSpec((B, 1), lambda i: (i, 0)),
            pl.BlockSpec((1, H), lambda i: (0, 0)),
            pl.BlockSpec((H, 1), lambda i: (0, 0)),
            pl.BlockSpec((1, 1), lambda i: (0, 0)),
            pl.BlockSpec((H, 1), lambda i: (0, 0)),
            pl.BlockSpec((1, 1), lambda i: (0, 0)),
            pl.BlockSpec((H, 2), lambda i: (0, 0)),
            pl.BlockSpec((1, 2), lambda i: (0, 0)),
            pl.BlockSpec((H, 9), lambda i: (0, 0)),
            pl.BlockSpec((1, 9), lambda i: (0, 0)),
        ],
        out_specs=[
            pl.BlockSpec((B, 1), lambda i: (i, 0)),
            pl.BlockSpec((B, 1), lambda i: (i, 0)),
            pl.BlockSpec((1, 2), lambda i: (0, 0)),
            pl.BlockSpec((1, 9), lambda i: (0, 0)),
        ],
        out_shape=[
            jax.ShapeDtypeStruct((N, 1), jnp.float32),
            jax.ShapeDtypeStruct((N, 1), jnp.float32),
            jax.ShapeDtypeStruct((1, 2), jnp.float32),
            jax.ShapeDtypeStruct((1, 9), jnp.float32),
        ],
        scratch_shapes=[
            pltpu.SMEM((1,), jnp.float32),
            pltpu.VMEM((1, H), jnp.float32),
        ],
    )(s0, s1, dinv, b3, Wd, bd, Wa, ba, Wt, bt, Wact, bact)


def kernel(x, edge_index, W1, b1, W2, b2, W3, b3, Wd, bd, Wa, ba,
           Wt, bt, Wact, bact):
    rowd = edge_index[0].reshape(SROWS, MICRO)
    col2 = edge_index[1].reshape(NROWS, ROWW)
    cold = edge_index[1].reshape(SROWS, MICRO)

    deg0, deg1 = _deg_call(col2)
    g0, g1, dinv = _tc1(x, W1, deg0.reshape(N, 1), deg1.reshape(N, 1))
    s0, s1 = _spmm_call(rowd, cold, g0, g1)
    g0, g1 = _tcmid(s0, s1, dinv, b1.reshape(1, H), W2)
    s0, s1 = _spmm_call(rowd, cold, g0, g1)
    g0, g1 = _tcmid(s0, s1, dinv, b2.reshape(1, H), W3)
    s0, s1 = _spmm_call(rowd, cold, g0, g1)
    dist, att, tls, al = _heads(
        s0, s1, dinv, b3.reshape(1, H), Wd, bd.reshape(1, 1),
        Wa, ba.reshape(1, 1), Wt, bt.reshape(1, 2), Wact, bact.reshape(1, 9))

    return (dist.reshape(N), att.reshape(N), tls.reshape(2), al.reshape(9))


# speedup vs baseline: 1.0429x; 1.0429x over previous
import jax
import jax.numpy as jnp
from jax import lax
from jax.experimental import pallas as pl
from jax.experimental.pallas import tpu as pltpu
from jax.experimental.pallas import tpu_sc as plsc

N = 100000
E = 3200000
D_IN = 5
H = 32
HH = 16

ROWW = 100
NROWS = E // ROWW
NS = 16
CHUNK = 8
DPT = NROWS // 32
DCH = DPT // CHUNK
MICRO = 100
SROWS = E // MICRO
SPT = SROWS // NS
IDXB = 40
NOUT = SPT // IDXB
NCHI = IDXB
NBUF = 4
ACC_SLICE = 6256
ACC_LAST = N - 15 * ACC_SLICE
WOUT = 112


def _acc_chunks(total):
    offs = []
    o = 0
    while o < total:
        sz = min(WOUT, total - o)
        offs.append((o, sz))
        o += sz
    return offs
DEG_SLICE = 6240
DEG_LAST = N - 15 * DEG_SLICE

B = 2000
NB = N // B


def _sc_deg_body(col2, deg0, deg1, colv, onesv, dbuf, dacc):
    cid = lax.axis_index("c")
    sid = lax.axis_index("s")

    def fill_ones(i, c):
        onesv[pl.ds(i * 16, 16)] = jnp.full((16,), 1.0, jnp.float32)
        return c

    lax.fori_loop(0, 7, fill_ones, 0)

    def fill_zero(i, c):
        dbuf[pl.ds(i * 16, 16)] = jnp.zeros((16,), jnp.float32)
        return c

    lax.fori_loop(0, DEG_LAST // 16, fill_zero, 0)

    @pl.when(sid < 15)
    def _():
        pltpu.sync_copy(dbuf.at[pl.ds(0, DEG_SLICE)],
                        dacc.at[pl.ds(sid * DEG_SLICE, DEG_SLICE)])

    @pl.when(sid == 15)
    def _():
        pltpu.sync_copy(dbuf, dacc.at[pl.ds(15 * DEG_SLICE, DEG_LAST)])

    plsc.subcore_barrier()

    w = cid * NS + sid

    def chunk_body(ch, c):
        base = w * DPT + ch * CHUNK
        pltpu.sync_copy(col2.at[pl.ds(base, CHUNK)], colv)
        for j in range(CHUNK):
            pltpu.sync_copy(onesv.at[pl.ds(0, ROWW)],
                            dacc.at[colv.at[j]], add=True)
        return c

    lax.fori_loop(0, DCH, chunk_body, 0)
    plsc.subcore_barrier()

    def writeout(dst):
        @pl.when(sid < 15)
        def _():
            pltpu.sync_copy(dacc.at[pl.ds(sid * DEG_SLICE, DEG_SLICE)],
                            dbuf.at[pl.ds(0, DEG_SLICE)])
            pltpu.sync_copy(dbuf.at[pl.ds(0, DEG_SLICE)],
                            dst.at[pl.ds(sid * DEG_SLICE, DEG_SLICE)])

        @pl.when(sid == 15)
        def _():
            pltpu.sync_copy(dacc.at[pl.ds(15 * DEG_SLICE, DEG_LAST)], dbuf)
            pltpu.sync_copy(dbuf, dst.at[pl.ds(15 * DEG_SLICE, DEG_LAST)])

    @pl.when(cid == 0)
    def _():
        writeout(deg0)

    @pl.when(cid == 1)
    def _():
        writeout(deg1)


def _sc_spmm_body(row2, col2, g0, g1, s0, s1, rowv, colv, rows2, wbuf, acc,
                  sg0, sg1, sg2, sg3, ss0, ss1, ss2, ss3):
    cid = lax.axis_index("c")
    sid = lax.axis_index("s")

    def fill_zero(i, c):
        wbuf[i, :] = jnp.zeros((16,), jnp.float32)
        return c

    lax.fori_loop(0, WOUT, fill_zero, 0)

    base = sid * ACC_SLICE

    @pl.when(sid < 15)
    def _():
        for o, sz in _acc_chunks(ACC_SLICE):
            pltpu.sync_copy(wbuf.at[pl.ds(0, sz)],
                            acc.at[pl.ds(base + o, sz)])

    @pl.when(sid == 15)
    def _():
        for o, sz in _acc_chunks(ACC_LAST):
            pltpu.sync_copy(wbuf.at[pl.ds(0, sz)],
                            acc.at[pl.ds(base + o, sz)])

    plsc.subcore_barrier()

    sg = (sg0, sg1, sg2, sg3)
    ss = (ss0, ss1, ss2, ss3)

    def fire_g(gref, c, b):
        pltpu.async_copy(gref.at[rowv.at[c]], rows2.at[b], sg[b])

    def drain_g(gref, b):
        pltpu.make_async_copy(gref.at[pl.ds(0, MICRO)],
                              rows2.at[b], sg[b]).wait()

    def fire_s(c, b):
        pltpu.async_copy(rows2.at[b], acc.at[colv.at[c]], ss[b], add=True)

    def drain_s(b):
        pltpu.make_async_copy(rows2.at[b],
                              acc.at[pl.ds(0, MICRO)], ss[b]).wait()

    def outer(t, c_):
        obase = sid * SPT + t * IDXB
        pltpu.sync_copy(row2.at[pl.ds(obase, IDXB)], rowv)
        pltpu.sync_copy(col2.at[pl.ds(obase, IDXB)], colv)

        def run(gref):
            for k in range(NBUF - 1):
                fire_g(gref, k, k)
            for c in range(NCHI):
                b = c % NBUF
                nf = c + NBUF - 1
                if nf < NCHI:
                    bn = nf % NBUF
                    if nf >= NBUF:
                        drain_s(bn)
                    fire_g(gref, nf, bn)
                drain_g(gref, b)
                fire_s(c, b)
            for b in range(NBUF):
                drain_s(b)

        @pl.when(cid == 0)
        def _():
            run(g0)

        @pl.when(cid == 1)
        def _():
            run(g1)
        return c_

    lax.fori_loop(0, NOUT, outer, 0)
    plsc.subcore_barrier()

    def writeout(dst):
        @pl.when(sid < 15)
        def _():
            for o, sz in _acc_chunks(ACC_SLICE):
                pltpu.sync_copy(acc.at[pl.ds(base + o, sz)],
                                wbuf.at[pl.ds(0, sz)])
                pltpu.sync_copy(wbuf.at[pl.ds(0, sz)],
                                dst.at[pl.ds(base + o, sz)])

        @pl.when(sid == 15)
        def _():
            for o, sz in _acc_chunks(ACC_LAST):
                pltpu.sync_copy(acc.at[pl.ds(base + o, sz)],
                                wbuf.at[pl.ds(0, sz)])
                pltpu.sync_copy(wbuf.at[pl.ds(0, sz)],
                                dst.at[pl.ds(base + o, sz)])

    @pl.when(cid == 0)
    def _():
        writeout(s0)

    @pl.when(cid == 1)
    def _():
        writeout(s1)


def _sc_mesh():
    return plsc.VectorSubcoreMesh(core_axis_name="c", subcore_axis_name="s")


_SC_PARAMS = pltpu.CompilerParams(use_tc_tiling_on_sc=False)


def _deg_call(col2):
    f = pl.kernel(
        _sc_deg_body,
        out_type=(jax.ShapeDtypeStruct((N,), jnp.float32),
                  jax.ShapeDtypeStruct((N,), jnp.float32)),
        mesh=_sc_mesh(),
        compiler_params=_SC_PARAMS,
        scratch_types=[
            pltpu.VMEM((CHUNK, ROWW), jnp.int32),
            pltpu.VMEM((112,), jnp.float32),
            pltpu.VMEM((DEG_LAST,), jnp.float32),
            pltpu.VMEM_SHARED((N,), jnp.float32),
        ],
    )
    return f(col2)


def _spmm_call(row2, col2, g0, g1):
    f = pl.kernel(
        _sc_spmm_body,
        out_type=(jax.ShapeDtypeStruct((N, HH), jnp.float32),
                  jax.ShapeDtypeStruct((N, HH), jnp.float32)),
        mesh=_sc_mesh(),
        compiler_params=_SC_PARAMS,
        scratch_types=[
            pltpu.VMEM((IDXB, MICRO), jnp.int32),
            pltpu.VMEM((IDXB, MICRO), jnp.int32),
            pltpu.VMEM((NBUF, MICRO, HH), jnp.float32),
            pltpu.VMEM((WOUT, HH), jnp.float32),
            pltpu.VMEM_SHARED((N, HH), jnp.float32),
        ] + [pltpu.SemaphoreType.DMA] * (2 * NBUF),
    )
    return f(row2, col2, g0, g1)


def _tc1_body(x_ref, w1_ref, d0_ref, d1_ref, g0_ref, g1_ref, dinv_ref):
    deg = d0_ref[...] + d1_ref[...]
    dinv = jnp.where(deg > 0.0, lax.rsqrt(jnp.maximum(deg, 1e-12)), 0.0)
    g = jnp.dot(x_ref[...], w1_ref[...],
                preferred_element_type=jnp.float32) * dinv
    g0_ref[...] = g[:, :HH]
    g1_ref[...] = g[:, HH:]
    dinv_ref[...] = dinv


def _tc1(x, W1, d0, d1):
    return pl.pallas_call(
        _tc1_body,
        grid=(NB,),
        in_specs=[
            pl.BlockSpec((B, D_IN), lambda i: (i, 0)),
            pl.BlockSpec((D_IN, H), lambda i: (0, 0)),
            pl.BlockSpec((B, 1), lambda i: (i, 0)),
            pl.BlockSpec((B, 1), lambda i: (i, 0)),
        ],
        out_specs=[
            pl.BlockSpec((B, HH), lambda i: (i, 0)),
            pl.BlockSpec((B, HH), lambda i: (i, 0)),
            pl.BlockSpec((B, 1), lambda i: (i, 0)),
        ],
        out_shape=[
            jax.ShapeDtypeStruct((N, HH), jnp.float32),
            jax.ShapeDtypeStruct((N, HH), jnp.float32),
            jax.ShapeDtypeStruct((N, 1), jnp.float32),
        ],
    )(x, W1, d0, d1)


def _tcmid_body(s0_ref, s1_ref, dinv_ref, b_ref, w_ref, g0_ref, g1_ref):
    dinv = dinv_ref[...]
    h = jnp.concatenate([s0_ref[...], s1_ref[...]], axis=1) * dinv + b_ref[...]
    h = jnp.maximum(h, 0.0)
    g = jnp.dot(h, w_ref[...], preferred_element_type=jnp.float32) * dinv
    g0_ref[...] = g[:, :HH]
    g1_ref[...] = g[:, HH:]


def _tcmid(s0, s1, dinv, b, W):
    return pl.pallas_call(
        _tcmid_body,
        grid=(NB,),
        in_specs=[
            pl.BlockSpec((B, HH), lambda i: (i, 0)),
            pl.BlockSpec((B, HH), lambda i: (i, 0)),
            pl.BlockSpec((B, 1), lambda i: (i, 0)),
            pl.BlockSpec((1, H), lambda i: (0, 0)),
            pl.BlockSpec((H, H), lambda i: (0, 0)),
        ],
        out_specs=[
            pl.BlockSpec((B, HH), lambda i: (i, 0)),
            pl.BlockSpec((B, HH), lambda i: (i, 0)),
        ],
        out_shape=[
            jax.ShapeDtypeStruct((N, HH), jnp.float32),
            jax.ShapeDtypeStruct((N, HH), jnp.float32),
        ],
    )(s0, s1, dinv, b, W)


def _heads_body(s0_ref, s1_ref, dinv_ref, b3_ref, wd_ref, bd_ref, wa_ref,
                ba_ref, wt_ref, bt_ref, wact_ref, bact_ref,
                dist_ref, att_ref, tls_ref, al_ref, best_val, best_row):
    i = pl.program_id(0)

    @pl.when(i == 0)
    def _():
        best_val[0] = -jnp.inf

    dinv = dinv_ref[...]
    h = jnp.concatenate([s0_ref[...], s1_ref[...]], axis=1) * dinv + b3_ref[...]
    h = jnp.maximum(h, 0.0)
    d = jnp.dot(h, wd_ref[...], preferred_element_type=jnp.float32) + bd_ref[...]
    a = jnp.dot(h, wa_ref[...], preferred_element_type=jnp.float32) + ba_ref[...]
    dist_ref[...] = d
    att_ref[...] = a

    loc_max = jnp.max(a)
    ids = lax.broadcasted_iota(jnp.int32, (B, 1), 0)
    loc_arg = jnp.min(jnp.where(a == loc_max, ids, N))

    @pl.when(loc_max > best_val[0])
    def _():
        best_val[0] = loc_max
        best_row[...] = jnp.sum(h * (ids == loc_arg).astype(jnp.float32),
                                axis=0, keepdims=True)

    @pl.when(i == NB - 1)
    def _():
        ht = best_row[...]
        tls_ref[...] = jnp.dot(ht, wt_ref[...],
                               preferred_element_type=jnp.float32) + bt_ref[...]
        al_ref[...] = jnp.dot(ht, wact_ref[...],
                              preferred_element_type=jnp.float32) + bact_ref[...]


def _heads(s0, s1, dinv, b3, Wd, bd, Wa, ba, Wt, bt, Wact, bact):
    return pl.pallas_call(
        _heads_body,
        grid=(NB,),
        in_specs=[
            pl.BlockSpec((B, HH), lambda i: (i, 0)),
            pl.BlockSpec((B, HH), lambda i: (i, 0)),
            pl.BlockSpec((B, 1), lambda i: (i, 0)),
            pl.BlockSpec((1, H), lambda i: (0, 0)),
            pl.BlockSpec((H, 1), lambda i: (0, 0)),
            pl.BlockSpec((1, 1), lambda i: (0, 0)),
            pl.BlockSpec((H, 1), lambda i: (0, 0)),
            pl.BlockSpec((1, 1), lambda i: (0, 0)),
            pl.BlockSpec((H, 2), lambda i: (0, 0)),
            pl.BlockSpec((1, 2), lambda i: (0, 0)),
            pl.BlockSpec((H, 9), lambda i: (0, 0)),
            pl.BlockSpec((1, 9), lambda i: (0, 0)),
        ],
        out_specs=[
            pl.BlockSpec((B, 1), lambda i: (i, 0)),
            pl.BlockSpec((B, 1), lambda i: (i, 0)),
            pl.BlockSpec((1, 2), lambda i: (0, 0)),
            pl.BlockSpec((1, 9), lambda i: (0, 0)),
        ],
        out_shape=[
            jax.ShapeDtypeStruct((N, 1), jnp.float32),
            jax.ShapeDtypeStruct((N, 1), jnp.float32),
            jax.ShapeDtypeStruct((1, 2), jnp.float32),
            jax.ShapeDtypeStruct((1, 9), jnp.float32),
        ],
        scratch_shapes=[
            pltpu.SMEM((1,), jnp.float32),
            pltpu.VMEM((1, H), jnp.float32),
        ],
    )(s0, s1, dinv, b3, Wd, bd, Wa, ba, Wt, bt, Wact, bact)


def kernel(x, edge_index, W1, b1, W2, b2, W3, b3, Wd, bd, Wa, ba,
           Wt, bt, Wact, bact):
    rowd = edge_index[0].reshape(SROWS, MICRO)
    col2 = edge_index[1].reshape(NROWS, ROWW)
    cold = edge_index[1].reshape(SROWS, MICRO)

    deg0, deg1 = _deg_call(col2)
    g0, g1, dinv = _tc1(x, W1, deg0.reshape(N, 1), deg1.reshape(N, 1))
    s0, s1 = _spmm_call(rowd, cold, g0, g1)
    g0, g1 = _tcmid(s0, s1, dinv, b1.reshape(1, H), W2)
    s0, s1 = _spmm_call(rowd, cold, g0, g1)
    g0, g1 = _tcmid(s0, s1, dinv, b2.reshape(1, H), W3)
    s0, s1 = _spmm_call(rowd, cold, g0, g1)
    dist, att, tls, al = _heads(
        s0, s1, dinv, b3.reshape(1, H), Wd, bd.reshape(1, 1),
        Wa, ba.reshape(1, 1), Wt, bt.reshape(1, 2), Wact, bact.reshape(1, 9))

    return (dist.reshape(N), att.reshape(N), tls.reshape(2), al.reshape(9))


# speedup vs baseline: 1.2408x; 1.1898x over previous
import jax
import jax.numpy as jnp
from jax import lax
from jax.experimental import pallas as pl
from jax.experimental.pallas import tpu as pltpu
from jax.experimental.pallas import tpu_sc as plsc

N = 100000
E = 3200000
D_IN = 5
H = 32
HH = 16

ROWW = 100
NROWS = E // ROWW
NS = 16
CHUNK = 8
DPT = NROWS // 32
DCH = DPT // CHUNK
MICRO = 100
SROWS = E // MICRO
SPT = SROWS // NS
IDXB = 40
NOUT = SPT // IDXB
CH_ROWS = 4
NCHI = IDXB // CH_ROWS
NBUF = 3
ACC_SLICE = 6256
ACC_LAST = N - 15 * ACC_SLICE
WOUT = 112


def _acc_chunks(total):
    offs = []
    o = 0
    while o < total:
        sz = min(WOUT, total - o)
        offs.append((o, sz))
        o += sz
    return offs
DEG_SLICE = 6240
DEG_LAST = N - 15 * DEG_SLICE

B = 2000
NB = N // B


def _sc_deg_body(col2, deg0, deg1, colv, onesv, dbuf, dacc):
    cid = lax.axis_index("c")
    sid = lax.axis_index("s")

    def fill_ones(i, c):
        onesv[pl.ds(i * 16, 16)] = jnp.full((16,), 1.0, jnp.float32)
        return c

    lax.fori_loop(0, 7, fill_ones, 0)

    def fill_zero(i, c):
        dbuf[pl.ds(i * 16, 16)] = jnp.zeros((16,), jnp.float32)
        return c

    lax.fori_loop(0, DEG_LAST // 16, fill_zero, 0)

    @pl.when(sid < 15)
    def _():
        pltpu.sync_copy(dbuf.at[pl.ds(0, DEG_SLICE)],
                        dacc.at[pl.ds(sid * DEG_SLICE, DEG_SLICE)])

    @pl.when(sid == 15)
    def _():
        pltpu.sync_copy(dbuf, dacc.at[pl.ds(15 * DEG_SLICE, DEG_LAST)])

    plsc.subcore_barrier()

    w = cid * NS + sid

    def chunk_body(ch, c):
        base = w * DPT + ch * CHUNK
        pltpu.sync_copy(col2.at[pl.ds(base, CHUNK)], colv)
        for j in range(CHUNK):
            pltpu.sync_copy(onesv.at[pl.ds(0, ROWW)],
                            dacc.at[colv.at[j]], add=True)
        return c

    lax.fori_loop(0, DCH, chunk_body, 0)
    plsc.subcore_barrier()

    def writeout(dst):
        @pl.when(sid < 15)
        def _():
            pltpu.sync_copy(dacc.at[pl.ds(sid * DEG_SLICE, DEG_SLICE)],
                            dbuf.at[pl.ds(0, DEG_SLICE)])
            pltpu.sync_copy(dbuf.at[pl.ds(0, DEG_SLICE)],
                            dst.at[pl.ds(sid * DEG_SLICE, DEG_SLICE)])

        @pl.when(sid == 15)
        def _():
            pltpu.sync_copy(dacc.at[pl.ds(15 * DEG_SLICE, DEG_LAST)], dbuf)
            pltpu.sync_copy(dbuf, dst.at[pl.ds(15 * DEG_SLICE, DEG_LAST)])

    @pl.when(cid == 0)
    def _():
        writeout(deg0)

    @pl.when(cid == 1)
    def _():
        writeout(deg1)


def _sc_spmm_body(row2, col2, g0, g1, s0, s1, rowv, colv, rows2, wbuf, acc,
                  sg0, sg1, sg2, ss0, ss1, ss2):
    cid = lax.axis_index("c")
    sid = lax.axis_index("s")

    def fill_zero(i, c):
        wbuf[i, :] = jnp.zeros((16,), jnp.float32)
        return c

    lax.fori_loop(0, WOUT, fill_zero, 0)

    base = sid * ACC_SLICE

    @pl.when(sid < 15)
    def _():
        for o, sz in _acc_chunks(ACC_SLICE):
            pltpu.sync_copy(wbuf.at[pl.ds(0, sz)],
                            acc.at[pl.ds(base + o, sz)])

    @pl.when(sid == 15)
    def _():
        for o, sz in _acc_chunks(ACC_LAST):
            pltpu.sync_copy(wbuf.at[pl.ds(0, sz)],
                            acc.at[pl.ds(base + o, sz)])

    plsc.subcore_barrier()

    sg = (sg0, sg1, sg2)
    ss = (ss0, ss1, ss2)

    def fire_g(gref, c, b):
        for j in range(CH_ROWS):
            pltpu.async_copy(gref.at[rowv.at[c * CH_ROWS + j]],
                             rows2.at[b].at[pl.ds(j * MICRO, MICRO)], sg[b])

    def drain_g(gref, b):
        pltpu.make_async_copy(gref.at[pl.ds(0, CH_ROWS * MICRO)],
                              rows2.at[b], sg[b]).wait()

    def fire_s(c, b):
        for j in range(CH_ROWS):
            pltpu.async_copy(rows2.at[b].at[pl.ds(j * MICRO, MICRO)],
                             acc.at[colv.at[c * CH_ROWS + j]], ss[b],
                             add=True)

    def drain_s(b):
        pltpu.make_async_copy(rows2.at[b],
                              acc.at[pl.ds(0, CH_ROWS * MICRO)], ss[b]).wait()

    def outer(t, c_):
        obase = sid * SPT + t * IDXB
        pltpu.sync_copy(row2.at[pl.ds(obase, IDXB)], rowv)
        pltpu.sync_copy(col2.at[pl.ds(obase, IDXB)], colv)

        def run(gref):
            for k in range(NBUF - 1):
                fire_g(gref, k, k)
            for c in range(NCHI):
                b = c % NBUF
                nf = c + NBUF - 1
                if nf < NCHI:
                    bn = nf % NBUF
                    if nf >= NBUF:
                        drain_s(bn)
                    fire_g(gref, nf, bn)
                drain_g(gref, b)
                fire_s(c, b)
            for b in range(NBUF):
                drain_s(b)

        @pl.when(cid == 0)
        def _():
            run(g0)

        @pl.when(cid == 1)
        def _():
            run(g1)
        return c_

    lax.fori_loop(0, NOUT, outer, 0)
    plsc.subcore_barrier()

    def writeout(dst):
        @pl.when(sid < 15)
        def _():
            for o, sz in _acc_chunks(ACC_SLICE):
                pltpu.sync_copy(acc.at[pl.ds(base + o, sz)],
                                wbuf.at[pl.ds(0, sz)])
                pltpu.sync_copy(wbuf.at[pl.ds(0, sz)],
                                dst.at[pl.ds(base + o, sz)])

        @pl.when(sid == 15)
        def _():
            for o, sz in _acc_chunks(ACC_LAST):
                pltpu.sync_copy(acc.at[pl.ds(base + o, sz)],
                                wbuf.at[pl.ds(0, sz)])
                pltpu.sync_copy(wbuf.at[pl.ds(0, sz)],
                                dst.at[pl.ds(base + o, sz)])

    @pl.when(cid == 0)
    def _():
        writeout(s0)

    @pl.when(cid == 1)
    def _():
        writeout(s1)


def _sc_mesh():
    return plsc.VectorSubcoreMesh(core_axis_name="c", subcore_axis_name="s")


_SC_PARAMS = pltpu.CompilerParams(use_tc_tiling_on_sc=False)


def _deg_call(col2):
    f = pl.kernel(
        _sc_deg_body,
        out_type=(jax.ShapeDtypeStruct((N,), jnp.float32),
                  jax.ShapeDtypeStruct((N,), jnp.float32)),
        mesh=_sc_mesh(),
        compiler_params=_SC_PARAMS,
        scratch_types=[
            pltpu.VMEM((CHUNK, ROWW), jnp.int32),
            pltpu.VMEM((112,), jnp.float32),
            pltpu.VMEM((DEG_LAST,), jnp.float32),
            pltpu.VMEM_SHARED((N,), jnp.float32),
        ],
    )
    return f(col2)


def _spmm_call(row2, col2, g0, g1):
    f = pl.kernel(
        _sc_spmm_body,
        out_type=(jax.ShapeDtypeStruct((N, HH), jnp.float32),
                  jax.ShapeDtypeStruct((N, HH), jnp.float32)),
        mesh=_sc_mesh(),
        compiler_params=_SC_PARAMS,
        scratch_types=[
            pltpu.VMEM((IDXB, MICRO), jnp.int32),
            pltpu.VMEM((IDXB, MICRO), jnp.int32),
            pltpu.VMEM((NBUF, CH_ROWS * MICRO, HH), jnp.float32),
            pltpu.VMEM((WOUT, HH), jnp.float32),
            pltpu.VMEM_SHARED((N, HH), jnp.float32),
        ] + [pltpu.SemaphoreType.DMA] * (2 * NBUF),
    )
    return f(row2, col2, g0, g1)


def _tc1_body(x_ref, w1_ref, d0_ref, d1_ref, g0_ref, g1_ref, dinv_ref):
    deg = d0_ref[...] + d1_ref[...]
    dinv = jnp.where(deg > 0.0, lax.rsqrt(jnp.maximum(deg, 1e-12)), 0.0)
    g = jnp.dot(x_ref[...], w1_ref[...],
                preferred_element_type=jnp.float32) * dinv
    g0_ref[...] = g[:, :HH]
    g1_ref[...] = g[:, HH:]
    dinv_ref[...] = dinv


def _tc1(x, W1, d0, d1):
    return pl.pallas_call(
        _tc1_body,
        grid=(NB,),
        in_specs=[
            pl.BlockSpec((B, D_IN), lambda i: (i, 0)),
            pl.BlockSpec((D_IN, H), lambda i: (0, 0)),
            pl.BlockSpec((B, 1), lambda i: (i, 0)),
            pl.BlockSpec((B, 1), lambda i: (i, 0)),
        ],
        out_specs=[
            pl.BlockSpec((B, HH), lambda i: (i, 0)),
            pl.BlockSpec((B, HH), lambda i: (i, 0)),
            pl.BlockSpec((B, 1), lambda i: (i, 0)),
        ],
        out_shape=[
            jax.ShapeDtypeStruct((N, HH), jnp.float32),
            jax.ShapeDtypeStruct((N, HH), jnp.float32),
            jax.ShapeDtypeStruct((N, 1), jnp.float32),
        ],
    )(x, W1, d0, d1)


def _tcmid_body(s0_ref, s1_ref, dinv_ref, b_ref, w_ref, g0_ref, g1_ref):
    dinv = dinv_ref[...]
    h = jnp.concatenate([s0_ref[...], s1_ref[...]], axis=1) * dinv + b_ref[...]
    h = jnp.maximum(h, 0.0)
    g = jnp.dot(h, w_ref[...], preferred_element_type=jnp.float32) * dinv
    g0_ref[...] = g[:, :HH]
    g1_ref[...] = g[:, HH:]


def _tcmid(s0, s1, dinv, b, W):
    return pl.pallas_call(
        _tcmid_body,
        grid=(NB,),
        in_specs=[
            pl.BlockSpec((B, HH), lambda i: (i, 0)),
            pl.BlockSpec((B, HH), lambda i: (i, 0)),
            pl.BlockSpec((B, 1), lambda i: (i, 0)),
            pl.BlockSpec((1, H), lambda i: (0, 0)),
            pl.BlockSpec((H, H), lambda i: (0, 0)),
        ],
        out_specs=[
            pl.BlockSpec((B, HH), lambda i: (i, 0)),
            pl.BlockSpec((B, HH), lambda i: (i, 0)),
        ],
        out_shape=[
            jax.ShapeDtypeStruct((N, HH), jnp.float32),
            jax.ShapeDtypeStruct((N, HH), jnp.float32),
        ],
    )(s0, s1, dinv, b, W)


def _heads_body(s0_ref, s1_ref, dinv_ref, b3_ref, wd_ref, bd_ref, wa_ref,
                ba_ref, wt_ref, bt_ref, wact_ref, bact_ref,
                dist_ref, att_ref, tls_ref, al_ref, best_val, best_row):
    i = pl.program_id(0)

    @pl.when(i == 0)
    def _():
        best_val[0] = -jnp.inf

    dinv = dinv_ref[...]
    h = jnp.concatenate([s0_ref[...], s1_ref[...]], axis=1) * dinv + b3_ref[...]
    h = jnp.maximum(h, 0.0)
    d = jnp.dot(h, wd_ref[...], preferred_element_type=jnp.float32) + bd_ref[...]
    a = jnp.dot(h, wa_ref[...], preferred_element_type=jnp.float32) + ba_ref[...]
    dist_ref[...] = d
    att_ref[...] = a

    loc_max = jnp.max(a)
    ids = lax.broadcasted_iota(jnp.int32, (B, 1), 0)
    loc_arg = jnp.min(jnp.where(a == loc_max, ids, N))

    @pl.when(loc_max > best_val[0])
    def _():
        best_val[0] = loc_max
        best_row[...] = jnp.sum(h * (ids == loc_arg).astype(jnp.float32),
                                axis=0, keepdims=True)

    @pl.when(i == NB - 1)
    def _():
        ht = best_row[...]
        tls_ref[...] = jnp.dot(ht, wt_ref[...],
                               preferred_element_type=jnp.float32) + bt_ref[...]
        al_ref[...] = jnp.dot(ht, wact_ref[...],
                              preferred_element_type=jnp.float32) + bact_ref[...]


def _heads(s0, s1, dinv, b3, Wd, bd, Wa, ba, Wt, bt, Wact, bact):
    return pl.pallas_call(
        _heads_body,
        grid=(NB,),
        in_specs=[
            pl.BlockSpec((B, HH), lambda i: (i, 0)),
            pl.BlockSpec((B, HH), lambda i: (i, 0)),
            pl.BlockSpec((B, 1), lambda i: (i, 0)),
            pl.BlockSpec((1, H), lambda i: (0, 0)),
            pl.BlockSpec((H, 1), lambda i: (0, 0)),
            pl.BlockSpec((1, 1), lambda i: (0, 0)),
            pl.BlockSpec((H, 1), lambda i: (0, 0)),
            pl.BlockSpec((1, 1), lambda i: (0, 0)),
            pl.BlockSpec((H, 2), lambda i: (0, 0)),
            pl.BlockSpec((1, 2), lambda i: (0, 0)),
            pl.BlockSpec((H, 9), lambda i: (0, 0)),
            pl.BlockSpec((1, 9), lambda i: (0, 0)),
        ],
        out_specs=[
            pl.BlockSpec((B, 1), lambda i: (i, 0)),
            pl.BlockSpec((B, 1), lambda i: (i, 0)),
            pl.BlockSpec((1, 2), lambda i: (0, 0)),
            pl.BlockSpec((1, 9), lambda i: (0, 0)),
        ],
        out_shape=[
            jax.ShapeDtypeStruct((N, 1), jnp.float32),
            jax.ShapeDtypeStruct((N, 1), jnp.float32),
            jax.ShapeDtypeStruct((1, 2), jnp.float32),
            jax.ShapeDtypeStruct((1, 9), jnp.float32),
        ],
        scratch_shapes=[
            pltpu.SMEM((1,), jnp.float32),
            pltpu.VMEM((1, H), jnp.float32),
        ],
    )(s0, s1, dinv, b3, Wd, bd, Wa, ba, Wt, bt, Wact, bact)


def kernel(x, edge_index, W1, b1, W2, b2, W3, b3, Wd, bd, Wa, ba,
           Wt, bt, Wact, bact):
    rowd = edge_index[0].reshape(SROWS, MICRO)
    col2 = edge_index[1].reshape(NROWS, ROWW)
    cold = edge_index[1].reshape(SROWS, MICRO)

    deg0, deg1 = _deg_call(col2)
    g0, g1, dinv = _tc1(x, W1, deg0.reshape(N, 1), deg1.reshape(N, 1))
    s0, s1 = _spmm_call(rowd, cold, g0, g1)
    g0, g1 = _tcmid(s0, s1, dinv, b1.reshape(1, H), W2)
    s0, s1 = _spmm_call(rowd, cold, g0, g1)
    g0, g1 = _tcmid(s0, s1, dinv, b2.reshape(1, H), W3)
    s0, s1 = _spmm_call(rowd, cold, g0, g1)
    dist, att, tls, al = _heads(
        s0, s1, dinv, b3.reshape(1, H), Wd, bd.reshape(1, 1),
        Wa, ba.reshape(1, 1), Wt, bt.reshape(1, 2), Wact, bact.reshape(1, 9))

    return (dist.reshape(N), att.reshape(N), tls.reshape(2), al.reshape(9))
